# baseline (speedup 1.0000x reference)
import functools

import jax
import jax.numpy as jnp
from jax import lax
from jax.experimental import pallas as pl

_B, _N, _D, _K = 4, 2048, 512, 16
_RB = 256
_NEG = -3.0e38


def _scores_body(xb_ref, xf_ref, s_ref, sum_ref):
    r = pl.program_id(1)
    xb = xb_ref[0]
    xf = xf_ref[0]
    sqb = jnp.sum(xb * xb, axis=1)
    sqf = jnp.sum(xf * xf, axis=1)
    g = lax.dot_general(xb, xf, (((1,), (1,)), ((), ())),
                        preferred_element_type=jnp.float32)
    s = 2.0 * g - sqb[:, None] - sqf[None, :]
    rows = r * _RB + lax.broadcasted_iota(jnp.int32, (_RB, _N), 0)
    cols = lax.broadcasted_iota(jnp.int32, (_RB, _N), 1)
    s_ref[0] = jnp.where(rows == cols, _NEG, s)
    ps = jnp.sum(xb, axis=0)[None, None, :]

    @pl.when(r == 0)
    def _():
        sum_ref[...] = ps

    @pl.when(r != 0)
    def _():
        sum_ref[...] += ps


def _scores(x):
    nb = _N // _RB
    return pl.pallas_call(
        _scores_body,
        grid=(_B, nb),
        in_specs=[
            pl.BlockSpec((1, _RB, _D), lambda g, r: (g, r, 0)),
            pl.BlockSpec((1, _N, _D), lambda g, r: (g, 0, 0)),
        ],
        out_specs=[
            pl.BlockSpec((1, _RB, _N), lambda g, r: (g, r, 0)),
            pl.BlockSpec((1, 1, _D), lambda g, r: (g, 0, 0)),
        ],
        out_shape=[
            jax.ShapeDtypeStruct((_B, _N, _N), jnp.float32),
            jax.ShapeDtypeStruct((_B, 1, _D), jnp.float32),
        ],
    )(x, x)


def _maxabs_body(x_ref, sum_ref, o_ref):
    x = x_ref[0]
    mean = sum_ref[0] * (1.0 / _N)
    o_ref[...] = jnp.max(jnp.abs(x - mean)).reshape(1, 1, 1)


def _maxabs(x, sumx):
    return pl.pallas_call(
        _maxabs_body,
        grid=(_B,),
        in_specs=[
            pl.BlockSpec((1, _N, _D), lambda g: (g, 0, 0)),
            pl.BlockSpec((1, 1, _D), lambda g: (g, 0, 0)),
        ],
        out_specs=pl.BlockSpec((1, 1, 1), lambda g: (g, 0, 0)),
        out_shape=jax.ShapeDtypeStruct((_B, 1, 1), jnp.float32),
    )(x, sumx)


def _layer_body(do_norm, do_relu, h_ref, w_ref, a2_ref, v1_ref, v2_ref,
                hw_ref, st_ref):
    h = h_ref[0]
    if do_norm:
        mean = v1_ref[0] * (1.0 / _N)
        scale = 0.999999 / v2_ref[0, 0, 0]
        h = (h - mean) * scale
    else:
        h = h + v1_ref[0]
        if do_relu:
            h = jnp.maximum(h, 0.0)
    hw = jnp.dot(h, w_ref[...], preferred_element_type=jnp.float32)
    hw_ref[0] = hw
    st_ref[0] = jnp.dot(hw, a2_ref[...], preferred_element_type=jnp.float32)


def _layer(hin, W, a_s, a_d, v1, v2, do_norm, do_relu):
    nb = _N // _RB
    a2 = jnp.stack([a_s, a_d], axis=1)
    if do_norm:
        v1_spec = pl.BlockSpec((1, 1, _D), lambda g, r: (g, 0, 0))
        v2_spec = pl.BlockSpec((1, 1, 1), lambda g, r: (g, 0, 0))
    else:
        v1_spec = pl.BlockSpec((1, 1, _D), lambda g, r: (0, 0, 0))
        v2_spec = pl.BlockSpec((1, 1, 1), lambda g, r: (0, 0, 0))
    return pl.pallas_call(
        functools.partial(_layer_body, do_norm, do_relu),
        grid=(_B, nb),
        in_specs=[
            pl.BlockSpec((1, _RB, _D), lambda g, r: (g, r, 0)),
            pl.BlockSpec((_D, _D), lambda g, r: (0, 0)),
            pl.BlockSpec((_D, 2), lambda g, r: (0, 0)),
            v1_spec,
            v2_spec,
        ],
        out_specs=[
            pl.BlockSpec((1, _RB, _D), lambda g, r: (g, r, 0)),
            pl.BlockSpec((1, _RB, 2), lambda g, r: (g, r, 0)),
        ],
        out_shape=[
            jax.ShapeDtypeStruct((_B, _N, _D), jnp.float32),
            jax.ShapeDtypeStruct((_B, _N, 2), jnp.float32),
        ],
    )(hin, W, a2, v1, v2)


def _aggregate(hW, st, idx):
    s_, t_ = st[..., 0], st[..., 1]
    sg = jnp.take_along_axis(s_, idx.reshape(_B, -1), axis=1).reshape(_B, _N, _K)
    e_n = sg + t_[..., None]
    e_s = s_ + t_
    logits = jnp.concatenate([e_n, e_s[..., None]], axis=2)
    logits = jnp.where(logits >= 0, logits, 0.2 * logits)
    m = jnp.max(logits, axis=2, keepdims=True)
    ex = jnp.exp(logits - m)
    alpha = ex / jnp.sum(ex, axis=2, keepdims=True)
    msgs = jax.vmap(lambda h, i: h[i])(hW, idx)
    return (jnp.einsum("bnk,bnkd->bnd", alpha[..., :_K], msgs)
            + alpha[..., _K:] * hW)


def _head_body(a_ref, b_ref, wc_ref, bc_ref, o_ref):
    z = a_ref[0] + b_ref[...]
    pooled = jnp.max(z, axis=0)[None, :]
    o_ref[0] = (jnp.dot(pooled, wc_ref[...],
                        preferred_element_type=jnp.float32) + bc_ref[...])


def _head(A2, b2, Wc, bc):
    return pl.pallas_call(
        _head_body,
        grid=(_B,),
        in_specs=[
            pl.BlockSpec((1, _N, _D), lambda g: (g, 0, 0)),
            pl.BlockSpec((1, _D), lambda g: (0, 0)),
            pl.BlockSpec((_D, 2), lambda g: (0, 0)),
            pl.BlockSpec((1, 2), lambda g: (0, 0)),
        ],
        out_specs=pl.BlockSpec((1, 1, 2), lambda g: (g, 0, 0)),
        out_shape=jax.ShapeDtypeStruct((_B, 1, 2), jnp.float32),
    )(A2, b2[None, :], Wc, bc[None, :])[:, 0, :]


def kernel(x, W0, att_src0, att_dst0, b0, W1, att_src1, att_dst1, b1,
           W2, att_src2, att_dst2, b2, Wc, bc):
    S, sumx = _scores(x)
    mx = _maxabs(x, sumx)
    idx = lax.top_k(S, _K)[1]

    hW, st = _layer(x, W0, att_src0, att_dst0, sumx, mx, True, False)
    A = _aggregate(hW, st, idx)
    hW, st = _layer(A, W1, att_src1, att_dst1, b0[None, None, :],
                    jnp.zeros((1, 1, 1), jnp.float32), False, True)
    A = _aggregate(hW, st, idx)
    hW, st = _layer(A, W2, att_src2, att_dst2, b1[None, None, :],
                    jnp.zeros((1, 1, 1), jnp.float32), False, True)
    A = _aggregate(hW, st, idx)
    return _head(A, b2, Wc, bc)


# speedup vs baseline: 15.1552x; 15.1552x over previous
import functools

import jax
import jax.numpy as jnp
from jax import lax
from jax.experimental import pallas as pl
from jax.experimental.pallas import tpu as pltpu
from jax.experimental.pallas import tpu_sc as plsc

_B, _N, _D, _K = 4, 2048, 512, 16
_RB = 256
_NEG = -3.0e38


def _scores_body(xb_ref, xf_ref, s_ref, m_ref, sum_ref):
    r = pl.program_id(1)
    xb = xb_ref[0]
    xf = xf_ref[0]
    sqb = jnp.sum(xb * xb, axis=1)
    sqf = jnp.sum(xf * xf, axis=1)
    g = lax.dot_general(xb, xf, (((1,), (1,)), ((), ())),
                        preferred_element_type=jnp.float32)
    s = 2.0 * g - sqb[:, None] - sqf[None, :]
    rows = r * _RB + lax.broadcasted_iota(jnp.int32, (_RB, _N), 0)
    cols = lax.broadcasted_iota(jnp.int32, (_RB, _N), 1)
    s = jnp.where(rows == cols, _NEG, s)
    s_ref[0] = s
    m_ref[0] = jnp.max(s.reshape(_RB, 16, 128), axis=1)
    ps = jnp.sum(xb, axis=0)[None, None, :]

    @pl.when(r == 0)
    def _():
        sum_ref[...] = ps

    @pl.when(r != 0)
    def _():
        sum_ref[...] += ps


def _scores(x):
    nb = _N // _RB
    return pl.pallas_call(
        _scores_body,
        grid=(_B, nb),
        in_specs=[
            pl.BlockSpec((1, _RB, _D), lambda g, r: (g, r, 0)),
            pl.BlockSpec((1, _N, _D), lambda g, r: (g, 0, 0)),
        ],
        out_specs=[
            pl.BlockSpec((1, _RB, _N), lambda g, r: (g, r, 0)),
            pl.BlockSpec((1, _RB, 128), lambda g, r: (g, r, 0)),
            pl.BlockSpec((1, 1, _D), lambda g, r: (g, 0, 0)),
        ],
        out_shape=[
            jax.ShapeDtypeStruct((_B, _N, _N), jnp.float32),
            jax.ShapeDtypeStruct((_B, _N, 128), jnp.float32),
            jax.ShapeDtypeStruct((_B, 1, _D), jnp.float32),
        ],
    )(x, x)


def _maxabs_body(x_ref, sum_ref, o_ref):
    x = x_ref[0]
    mean = sum_ref[0] * (1.0 / _N)
    o_ref[...] = jnp.max(jnp.abs(x - mean)).reshape(1, 1, 1)


def _maxabs(x, sumx):
    return pl.pallas_call(
        _maxabs_body,
        grid=(_B,),
        in_specs=[
            pl.BlockSpec((1, _N, _D), lambda g: (g, 0, 0)),
            pl.BlockSpec((1, 1, _D), lambda g: (g, 0, 0)),
        ],
        out_specs=pl.BlockSpec((1, 1, 1), lambda g: (g, 0, 0)),
        out_shape=jax.ShapeDtypeStruct((_B, 1, 1), jnp.float32),
    )(x, sumx)


def _layer_body(do_norm, do_relu, h_ref, w_ref, a2_ref, v1_ref, v2_ref,
                hw_ref, st_ref):
    h = h_ref[0]
    if do_norm:
        mean = v1_ref[0] * (1.0 / _N)
        scale = 0.999999 / v2_ref[0, 0, 0]
        h = (h - mean) * scale
    else:
        h = h + v1_ref[0]
        if do_relu:
            h = jnp.maximum(h, 0.0)
    hw = jnp.dot(h, w_ref[...], preferred_element_type=jnp.float32)
    hw_ref[0] = hw
    st_ref[0] = jnp.dot(hw, a2_ref[...], preferred_element_type=jnp.float32)


def _layer(hin, W, a_s, a_d, v1, v2, do_norm, do_relu):
    nb = _N // _RB
    a2 = jnp.stack([a_s, a_d], axis=1)
    if do_norm:
        v1_spec = pl.BlockSpec((1, 1, _D), lambda g, r: (g, 0, 0))
        v2_spec = pl.BlockSpec((1, 1, 1), lambda g, r: (g, 0, 0))
    else:
        v1_spec = pl.BlockSpec((1, 1, _D), lambda g, r: (0, 0, 0))
        v2_spec = pl.BlockSpec((1, 1, 1), lambda g, r: (0, 0, 0))
    return pl.pallas_call(
        functools.partial(_layer_body, do_norm, do_relu),
        grid=(_B, nb),
        in_specs=[
            pl.BlockSpec((1, _RB, _D), lambda g, r: (g, r, 0)),
            pl.BlockSpec((_D, _D), lambda g, r: (0, 0)),
            pl.BlockSpec((_D, 2), lambda g, r: (0, 0)),
            v1_spec,
            v2_spec,
        ],
        out_specs=[
            pl.BlockSpec((1, _RB, _D), lambda g, r: (g, r, 0)),
            pl.BlockSpec((1, _RB, 2), lambda g, r: (g, r, 0)),
        ],
        out_shape=[
            jax.ShapeDtypeStruct((_B, _N, _D), jnp.float32),
            jax.ShapeDtypeStruct((_B, _N, 2), jnp.float32),
        ],
    )(hin, W, a2, v1, v2)


_NROWS = _B * _N
_NW = 32
_RPW = _NROWS // _NW


def _merge16(tk, ti, keys, ids):
    ks, vs = plsc.sort_key_val(keys, ids)
    rk = lax.rev(ks, (0,))
    ri = lax.rev(vs, (0,))
    keep = tk >= rk
    mk = jnp.where(keep, tk, rk)
    mi = jnp.where(keep, ti, ri)
    return plsc.sort_key_val(mk, mi)


def _topk_body(s_hbm, m_hbm, out_hbm, row0, row1, m0, m1, obuf,
               sem0, sem1):
    cid = lax.axis_index("c")
    sid = lax.axis_index("s")
    base = (sid * 2 + cid) * _RPW
    iota = lax.broadcasted_iota(jnp.int32, (16,), 0)
    neg = jnp.full((16,), _NEG, jnp.float32)
    zeros = jnp.zeros((16,), jnp.int32)
    rows = (row0, row1)
    ms = (m0, m1)
    sems = (sem0, sem1)

    def start_dma(r, b):
        pltpu.async_copy(s_hbm.at[r], rows[b], sems[b])
        pltpu.async_copy(m_hbm.at[r], ms[b], sems[b])

    def wait_dma(b):
        pltpu.make_async_copy(s_hbm.at[0], rows[b], sems[b]).wait()
        pltpu.make_async_copy(m_hbm.at[0], ms[b], sems[b]).wait()

    start_dma(base, 0)
    start_dma(base + 1, 1)

    def process_row(r, b):
        wait_dma(b)
        tk, ti = neg, zeros
        for c in range(8):
            keys = ms[b][pl.ds(c * 16, 16)]
            tk, ti = _merge16(tk, ti, keys, c * 16 + iota)
        clsv = ti
        tk, ti = neg, zeros
        for k in range(16):
            l = clsv[k]
            idx = l + iota * 128
            vals = plsc.load_gather(rows[b], [idx])
            tk, ti = _merge16(tk, ti, vals, idx)
        obuf[pl.ds(r * _K, _K)] = ti
        nxt = jnp.minimum(r + 2, _RPW - 1)
        start_dma(base + nxt, b)

    def pair(i, carry):
        process_row(2 * i, 0)
        process_row(2 * i + 1, 1)
        return carry

    lax.fori_loop(0, _RPW // 2, pair, 0)
    wait_dma(0)
    wait_dma(1)
    pltpu.sync_copy(obuf, out_hbm.at[pl.ds(base * _K, _RPW * _K)])


def _topk(S, M):
    s2d = S.reshape(_NROWS, _N)
    m2d = M.reshape(_NROWS, 128)
    mesh = plsc.VectorSubcoreMesh(core_axis_name="c", subcore_axis_name="s",
                                  num_cores=2, num_subcores=16)
    call = pl.kernel(
        _topk_body,
        out_type=jax.ShapeDtypeStruct((_NROWS * _K,), jnp.int32),
        mesh=mesh,
        compiler_params=pltpu.CompilerParams(needs_layout_passes=False),
        scratch_types=[
            pltpu.VMEM((_N,), jnp.float32),
            pltpu.VMEM((_N,), jnp.float32),
            pltpu.VMEM((128,), jnp.float32),
            pltpu.VMEM((128,), jnp.float32),
            pltpu.VMEM((_RPW * _K,), jnp.int32),
            pltpu.SemaphoreType.DMA,
            pltpu.SemaphoreType.DMA,
        ],
    )
    return call(s2d, m2d).reshape(_B, _N, _K)


_NPT = _NROWS // _NW
_OB = 64


def _agg_body(hw_hbm, s_hbm, t_hbm, idx_hbm, out_hbm,
              sbuf, tbuf, idxbuf, rows0, rows1, self0, self1, outbuf,
              gsem0, gsem1):
    cid = lax.axis_index("c")
    sid = lax.axis_index("s")
    wid = sid * 2 + cid
    base = wid * _NPT
    gbase = (wid // 8) * _N
    rows = (rows0, rows1)
    selfs = (self0, self1)
    gsems = (gsem0, gsem1)

    pltpu.sync_copy(s_hbm, sbuf.at[pl.ds(0, _NROWS)])
    pltpu.sync_copy(t_hbm.at[pl.ds(base, _NPT)], tbuf.at[pl.ds(0, _NPT)])
    pltpu.sync_copy(idx_hbm.at[pl.ds(base * _K, _NPT * _K)], idxbuf)

    def neighbors(n):
        return idxbuf[pl.ds(n * _K, _K)] + gbase

    def issue(n, p):
        idxv = neighbors(n)
        pltpu.async_copy(hw_hbm.at[idxv], rows[p], gsems[p])
        pltpu.async_copy(hw_hbm.at[base + n], selfs[p], gsems[p])

    issue(0, 0)
    issue(1, 1)

    def process(n, j, p):
        idxv = neighbors(n)
        sv = plsc.load_gather(sbuf, [idxv])
        t_i = tbuf[pl.ds(n, 16)][0]
        ln = sv + t_i
        ln = jnp.where(ln >= 0, ln, 0.2 * ln)
        ss = sbuf[pl.ds(base + n, 16)][0]
        ls = ss + t_i
        ls = jnp.where(ls >= 0, ls, 0.2 * ls)
        m = jnp.maximum(jnp.max(ln), ls)
        ex = jnp.exp(ln - m)
        exs = jnp.exp(jnp.broadcast_to(ls - m, (16,)))[0]
        invv = 1.0 / jnp.broadcast_to(jnp.sum(ex) + exs, (16,))
        alpha = ex * invv
        asf = exs * invv[0]
        pltpu.make_async_copy(hw_hbm.at[pl.ds(0, _K)], rows[p],
                              gsems[p]).wait()
        pltpu.make_async_copy(hw_hbm.at[0], selfs[p], gsems[p]).wait()
        def chunk(c, carry):
            acc = asf * selfs[p][pl.ds(c * 16, 16)]
            for k in range(_K):
                acc = acc + alpha[k] * rows[p][k, pl.ds(c * 16, 16)]
            outbuf[pl.ds(j * _D + c * 16, 16)] = acc
            return carry

        lax.fori_loop(0, _D // 16, chunk, 0)
        nxt = jnp.minimum(n + 2, _NPT - 1)
        issue(nxt, p)

    for sb in range(_NPT // _OB):
        def pair(i, carry):
            n = sb * _OB + 2 * i
            process(n, 2 * i, 0)
            process(n + 1, 2 * i + 1, 1)
            return carry
        lax.fori_loop(0, _OB // 2, pair, 0)
        pltpu.sync_copy(outbuf,
                        out_hbm.at[pl.ds((base + sb * _OB) * _D, _OB * _D)])

    pltpu.make_async_copy(hw_hbm.at[pl.ds(0, _K)], rows[0], gsems[0]).wait()
    pltpu.make_async_copy(hw_hbm.at[0], selfs[0], gsems[0]).wait()
    pltpu.make_async_copy(hw_hbm.at[pl.ds(0, _K)], rows[1], gsems[1]).wait()
    pltpu.make_async_copy(hw_hbm.at[0], selfs[1], gsems[1]).wait()


def _aggregate_sc(hW, st, idx):
    hw2 = hW.reshape(_NROWS, _D)
    s_ = st[..., 0].reshape(_NROWS)
    t_ = st[..., 1].reshape(_NROWS)
    idxf = idx.reshape(_NROWS * _K)
    mesh = plsc.VectorSubcoreMesh(core_axis_name="c", subcore_axis_name="s",
                                  num_cores=2, num_subcores=16)
    call = pl.kernel(
        _agg_body,
        out_type=jax.ShapeDtypeStruct((_NROWS * _D,), jnp.float32),
        mesh=mesh,
        compiler_params=pltpu.CompilerParams(needs_layout_passes=False),
        scratch_types=[
            pltpu.VMEM((_NROWS + 16,), jnp.float32),
            pltpu.VMEM((_NPT + 16,), jnp.float32),
            pltpu.VMEM((_NPT * _K,), jnp.int32),
            pltpu.VMEM((_K, _D), jnp.float32),
            pltpu.VMEM((_K, _D), jnp.float32),
            pltpu.VMEM((_D,), jnp.float32),
            pltpu.VMEM((_D,), jnp.float32),
            pltpu.VMEM((_OB * _D,), jnp.float32),
            pltpu.SemaphoreType.DMA,
            pltpu.SemaphoreType.DMA,
        ],
    )
    return call(hw2, s_, t_, idxf).reshape(_B, _N, _D)


def _head_body(a_ref, b_ref, wc_ref, bc_ref, o_ref):
    z = a_ref[0] + b_ref[...]
    pooled = jnp.max(z, axis=0)[None, :]
    o_ref[0] = (jnp.dot(pooled, wc_ref[...],
                        preferred_element_type=jnp.float32) + bc_ref[...])


def _head(A2, b2, Wc, bc):
    return pl.pallas_call(
        _head_body,
        grid=(_B,),
        in_specs=[
            pl.BlockSpec((1, _N, _D), lambda g: (g, 0, 0)),
            pl.BlockSpec((1, _D), lambda g: (0, 0)),
            pl.BlockSpec((_D, 2), lambda g: (0, 0)),
            pl.BlockSpec((1, 2), lambda g: (0, 0)),
        ],
        out_specs=pl.BlockSpec((1, 1, 2), lambda g: (g, 0, 0)),
        out_shape=jax.ShapeDtypeStruct((_B, 1, 2), jnp.float32),
    )(A2, b2[None, :], Wc, bc[None, :])[:, 0, :]


def kernel(x, W0, att_src0, att_dst0, b0, W1, att_src1, att_dst1, b1,
           W2, att_src2, att_dst2, b2, Wc, bc):
    S, M, sumx = _scores(x)
    mx = _maxabs(x, sumx)
    idx = _topk(S, M)

    hW, st = _layer(x, W0, att_src0, att_dst0, sumx, mx, True, False)
    A = _aggregate_sc(hW, st, idx)
    hW, st = _layer(A, W1, att_src1, att_dst1, b0[None, None, :],
                    jnp.zeros((1, 1, 1), jnp.float32), False, True)
    A = _aggregate_sc(hW, st, idx)
    hW, st = _layer(A, W2, att_src2, att_dst2, b1[None, None, :],
                    jnp.zeros((1, 1, 1), jnp.float32), False, True)
    A = _aggregate_sc(hW, st, idx)
    return _head(A, b2, Wc, bc)
